# Initial kernel scaffold; baseline (speedup 1.0000x reference)
import jax
import jax.numpy as jnp
from jax import lax
from jax.experimental import pallas as pl
from jax.experimental.pallas import tpu as pltpu
from jax.experimental.pallas import tpu_sc as plsc

N_RAYS = 1048576
RAY_DIM = 16
NUM_VIEWS = 1000
TABLE_COLS = 12

_info = plsc.get_sparse_core_info()
_NC, _NS, _L = _info.num_cores, _info.num_subcores, _info.num_lanes
_NW = _NC * _NS
_ROWS_PER_W = N_RAYS // _NW
_CHUNK = 2048
_N_CHUNKS = _ROWS_PER_W // _CHUNK


def _sc_body(rays_hbm, table_hbm, out9_hbm, out3_hbm,
             rays_v, table_v, out9_v, out3_v):
    wid = lax.axis_index("s") * _NC + lax.axis_index("c")
    base_w = wid * _ROWS_PER_W
    pltpu.sync_copy(table_hbm, table_v)
    lane = lax.iota(jnp.int32, _L)
    col14 = jnp.full((_L,), RAY_DIM - 2, jnp.int32)

    def chunk_body(ci, carry):
        base = base_w + ci * _CHUNK
        pltpu.sync_copy(rays_hbm.at[pl.ds(base, _CHUNK)], rays_v)

        def row_body(j, carry2):
            r16 = j * _L + lane
            colv = plsc.load_gather(rays_v, [r16, col14])
            ids = (colv + 0.5).astype(jnp.int32)
            for c in range(9):
                cvec = jnp.full((_L,), c, jnp.int32)
                v = plsc.load_gather(table_v, [ids, cvec])
                plsc.store_scatter(out9_v, [r16, cvec], v)
            for c in range(3):
                v = plsc.load_gather(table_v, [ids, jnp.full((_L,), 9 + c, jnp.int32)])
                plsc.store_scatter(out3_v, [r16, jnp.full((_L,), c, jnp.int32)], v)
            return carry2

        lax.fori_loop(0, _CHUNK // _L, row_body, 0)
        pltpu.sync_copy(out9_v, out9_hbm.at[pl.ds(base, _CHUNK)])
        pltpu.sync_copy(out3_v, out3_hbm.at[pl.ds(base, _CHUNK)])
        return carry

    lax.fori_loop(0, _N_CHUNKS, chunk_body, 0)


def kernel(rays, color_embedding):
    mesh = plsc.VectorSubcoreMesh(core_axis_name="c", subcore_axis_name="s")
    f = pl.kernel(
        _sc_body,
        out_type=(
            jax.ShapeDtypeStruct((N_RAYS, 9), jnp.float32),
            jax.ShapeDtypeStruct((N_RAYS, 3), jnp.float32),
        ),
        mesh=mesh,
        scratch_types=[
            pltpu.VMEM((_CHUNK, RAY_DIM), jnp.float32),
            pltpu.VMEM((NUM_VIEWS, TABLE_COLS), jnp.float32),
            pltpu.VMEM((_CHUNK, 9), jnp.float32),
            pltpu.VMEM((_CHUNK, 3), jnp.float32),
        ],
    )
    return f(rays, color_embedding)


# speedup vs baseline: 2.4942x; 2.4942x over previous
import jax
import jax.numpy as jnp
from jax import lax
from jax.experimental import pallas as pl
from jax.experimental.pallas import tpu as pltpu
from jax.experimental.pallas import tpu_sc as plsc

N_RAYS = 1048576
RAY_DIM = 16
NUM_VIEWS = 1000
TABLE_COLS = 12

_NC, _NS, _L = 2, 16, 16
_NW = _NC * _NS
_ROWS_PER_W = N_RAYS // _NW
_CHUNK = 2048
_N_CHUNKS = _ROWS_PER_W // _CHUNK


def _sc_body(rays_hbm, table_hbm, out9_hbm, out3_hbm,
             rays_v, table_v, out9_v, out3_v):
    wid = lax.axis_index("s") * _NC + lax.axis_index("c")
    base_w = wid * _ROWS_PER_W
    pltpu.sync_copy(table_hbm, table_v)
    lane = lax.iota(jnp.int32, _L)

    def chunk_body(ci, carry):
        base = base_w + ci * _CHUNK
        pltpu.sync_copy(rays_hbm.at[pl.ds(base * RAY_DIM, _CHUNK * RAY_DIM)],
                        rays_v)

        def row_body(j, carry2):
            r16 = j * _L + lane
            colv = plsc.load_gather(rays_v, [r16 * RAY_DIM + (RAY_DIM - 2)])
            ids = (colv + 0.5).astype(jnp.int32)
            tbase = ids * TABLE_COLS
            o9 = r16 * 9
            o3 = r16 * 3
            for c in range(9):
                v = plsc.load_gather(table_v, [tbase + c])
                plsc.store_scatter(out9_v, [o9 + c], v)
            for c in range(3):
                v = plsc.load_gather(table_v, [tbase + (9 + c)])
                plsc.store_scatter(out3_v, [o3 + c], v)
            return carry2

        lax.fori_loop(0, _CHUNK // _L, row_body, 0)
        pltpu.sync_copy(out9_v, out9_hbm.at[pl.ds(base * 9, _CHUNK * 9)])
        pltpu.sync_copy(out3_v, out3_hbm.at[pl.ds(base * 3, _CHUNK * 3)])
        return carry

    lax.fori_loop(0, _N_CHUNKS, chunk_body, 0)


def kernel(rays, color_embedding):
    mesh = plsc.VectorSubcoreMesh(
        core_axis_name="c", subcore_axis_name="s",
        num_cores=_NC, num_subcores=_NS)
    f = pl.kernel(
        _sc_body,
        out_type=(
            jax.ShapeDtypeStruct((N_RAYS * 9,), jnp.float32),
            jax.ShapeDtypeStruct((N_RAYS * 3,), jnp.float32),
        ),
        mesh=mesh,
        compiler_params=pltpu.CompilerParams(needs_layout_passes=False),
        scratch_types=[
            pltpu.VMEM((_CHUNK * RAY_DIM,), jnp.float32),
            pltpu.VMEM((NUM_VIEWS * TABLE_COLS,), jnp.float32),
            pltpu.VMEM((_CHUNK * 9,), jnp.float32),
            pltpu.VMEM((_CHUNK * 3,), jnp.float32),
        ],
    )
    out9, out3 = f(rays.reshape(-1), color_embedding.reshape(-1))
    return out9.reshape(N_RAYS, 9), out3.reshape(N_RAYS, 3)


# speedup vs baseline: 25.2760x; 10.1341x over previous
import jax
import jax.numpy as jnp
from jax import lax
from jax.experimental import pallas as pl
from jax.experimental.pallas import tpu as pltpu
from jax.experimental.pallas import tpu_sc as plsc

N_RAYS = 1048576
RAY_DIM = 16
NUM_VIEWS = 1000
TABLE_COLS = 12

_NC, _NS, _L = 2, 16, 16
_NW = _NC * _NS
_ROWS_PER_W = N_RAYS // _NW
_CHUNK = 2048
_N_CHUNKS = _ROWS_PER_W // _CHUNK


def _sc_body(rays_t_hbm, table_hbm, out9_t_hbm, out3_t_hbm,
             ids_v, table_v, col_v):
    wid = lax.axis_index("s") * _NC + lax.axis_index("c")
    base_w = wid * _ROWS_PER_W
    pltpu.sync_copy(table_hbm, table_v)

    def chunk_body(ci, carry):
        base = base_w + ci * _CHUNK
        pltpu.sync_copy(rays_t_hbm.at[pl.ds(RAY_DIM - 2, 1), pl.ds(base, _CHUNK)],
                        ids_v)

        def row_body(j, carry2):
            colv = ids_v[0, pl.ds(j * _L, _L)]
            ids = (colv + 0.5).astype(jnp.int32)
            for c in range(TABLE_COLS):
                v = plsc.load_gather(table_v, [ids + (c * NUM_VIEWS)])
                col_v[c, pl.ds(j * _L, _L)] = v
            return carry2

        lax.fori_loop(0, _CHUNK // _L, row_body, 0)
        for c in range(9):
            pltpu.sync_copy(col_v.at[pl.ds(c, 1), pl.ds(0, _CHUNK)],
                            out9_t_hbm.at[pl.ds(c, 1), pl.ds(base, _CHUNK)])
        for c in range(3):
            pltpu.sync_copy(col_v.at[pl.ds(9 + c, 1), pl.ds(0, _CHUNK)],
                            out3_t_hbm.at[pl.ds(c, 1), pl.ds(base, _CHUNK)])
        return carry

    lax.fori_loop(0, _N_CHUNKS, chunk_body, 0)


def kernel(rays, color_embedding):
    mesh = plsc.VectorSubcoreMesh(
        core_axis_name="c", subcore_axis_name="s",
        num_cores=_NC, num_subcores=_NS)
    f = pl.kernel(
        _sc_body,
        out_type=(
            jax.ShapeDtypeStruct((9, N_RAYS), jnp.float32),
            jax.ShapeDtypeStruct((3, N_RAYS), jnp.float32),
        ),
        mesh=mesh,
        compiler_params=pltpu.CompilerParams(needs_layout_passes=False),
        scratch_types=[
            pltpu.VMEM((1, _CHUNK), jnp.float32),
            pltpu.VMEM((NUM_VIEWS * TABLE_COLS,), jnp.float32),
            pltpu.VMEM((TABLE_COLS, _CHUNK), jnp.float32),
        ],
    )
    table_flat = color_embedding.T.reshape(NUM_VIEWS * TABLE_COLS)
    out9_t, out3_t = f(rays.T, table_flat)
    return out9_t.T, out3_t.T


# speedup vs baseline: 75.7671x; 2.9976x over previous
import jax
import jax.numpy as jnp
from jax import lax
from jax.experimental import pallas as pl
from jax.experimental.pallas import tpu as pltpu
from jax.experimental.pallas import tpu_sc as plsc

N_RAYS = 1048576
RAY_DIM = 16
NUM_VIEWS = 1000
TABLE_COLS = 12

_NC, _NS, _L = 2, 16, 16
_NW = _NC * _NS
_ROWS_PER_W = N_RAYS // _NW
_CHUNK = 2048
_N_CHUNKS = _ROWS_PER_W // _CHUNK


def _sc_body(rays_t_hbm, table_hbm, out9_t_hbm, out3_t_hbm,
             ids_v, table_v, col_v, isem0, isem1, osem0, osem1):
    wid = lax.axis_index("s") * _NC + lax.axis_index("c")
    base_w = wid * _ROWS_PER_W
    isems = (isem0, isem1)
    osems = (osem0, osem1)
    pltpu.sync_copy(table_hbm, table_v)

    def in_copy(ci, b):
        return pltpu.make_async_copy(
            rays_t_hbm.at[pl.ds(RAY_DIM - 2, 1),
                          pl.ds(base_w + ci * _CHUNK, _CHUNK)],
            ids_v.at[pl.ds(b, 1), pl.ds(0, _CHUNK)],
            isems[b])

    def out_copies(ci, b):
        base = base_w + ci * _CHUNK
        cps = []
        for c in range(9):
            cps.append(pltpu.make_async_copy(
                col_v.at[pl.ds(b * TABLE_COLS + c, 1), pl.ds(0, _CHUNK)],
                out9_t_hbm.at[pl.ds(c, 1), pl.ds(base, _CHUNK)],
                osems[b]))
        for c in range(3):
            cps.append(pltpu.make_async_copy(
                col_v.at[pl.ds(b * TABLE_COLS + 9 + c, 1), pl.ds(0, _CHUNK)],
                out3_t_hbm.at[pl.ds(c, 1), pl.ds(base, _CHUNK)],
                osems[b]))
        return cps

    in_copy(0, 0).start()

    def outer(g, carry):
        for b in range(2):
            ci = g * 2 + b

            @pl.when(ci + 1 < _N_CHUNKS)
            def _():
                in_copy(ci + 1, 1 - b).start()

            in_copy(ci, b).wait()

            @pl.when(g >= 1)
            def _():
                for cp in out_copies(ci - 2, b):
                    cp.wait()

            @plsc.parallel_loop(0, _CHUNK // _L, 1, unroll=4)
            def row_body(j):
                colv = ids_v[b, pl.ds(j * _L, _L)]
                ids = (colv + 0.5).astype(jnp.int32)
                for c in range(TABLE_COLS):
                    v = plsc.load_gather(table_v, [ids + (c * NUM_VIEWS)])
                    col_v[b * TABLE_COLS + c, pl.ds(j * _L, _L)] = v

            for cp in out_copies(ci, b):
                cp.start()
        return carry

    lax.fori_loop(0, _N_CHUNKS // 2, outer, 0)
    for b in range(2):
        for cp in out_copies(_N_CHUNKS - 2 + b, b):
            cp.wait()


def kernel(rays, color_embedding):
    mesh = plsc.VectorSubcoreMesh(
        core_axis_name="c", subcore_axis_name="s",
        num_cores=_NC, num_subcores=_NS)
    f = pl.kernel(
        _sc_body,
        out_type=(
            jax.ShapeDtypeStruct((9, N_RAYS), jnp.float32),
            jax.ShapeDtypeStruct((3, N_RAYS), jnp.float32),
        ),
        mesh=mesh,
        compiler_params=pltpu.CompilerParams(needs_layout_passes=False),
        scratch_types=[
            pltpu.VMEM((2, _CHUNK), jnp.float32),
            pltpu.VMEM((NUM_VIEWS * TABLE_COLS,), jnp.float32),
            pltpu.VMEM((2 * TABLE_COLS, _CHUNK), jnp.float32),
            pltpu.SemaphoreType.DMA,
            pltpu.SemaphoreType.DMA,
            pltpu.SemaphoreType.DMA,
            pltpu.SemaphoreType.DMA,
        ],
    )
    table_flat = color_embedding.T.reshape(NUM_VIEWS * TABLE_COLS)
    out9_t, out3_t = f(rays.T, table_flat)
    return out9_t.T, out3_t.T
